# baseline (speedup 1.0000x reference)
import functools

import jax
import jax.numpy as jnp
from jax import lax
from jax.experimental import pallas as pl
from jax.experimental.pallas import tpu as pltpu
from jax.experimental.pallas import tpu_sc as plsc

N = 10000
D = 128
HD = D // 2
E = 320000

NC = 2
NS = 16
NW = NC * NS
CHUNK = 128

EW = E // NW
SLABD = 512
JD = EW // SLABD
TW = EW - JD * SLABD

ET = E // NS
SLABA = 256
JA = ET // SLABA
TA = ET - JA * SLABA
NBUF = 3

SPAN = 640
N_ACC = NS * SPAN

_mesh = plsc.VectorSubcoreMesh(core_axis_name="c", subcore_axis_name="s")


@functools.partial(
    pl.kernel,
    out_type=jax.ShapeDtypeStruct((NC * N_ACC,), jnp.float32),
    mesh=_mesh,
    compiler_params=pltpu.CompilerParams(use_tc_tiling_on_sc=False),
    scratch_types=[
        pltpu.VMEM((EW,), jnp.int32),
        pltpu.VMEM((SLABD,), jnp.float32),
        pltpu.VMEM((SPAN,), jnp.float32),
        pltpu.VMEM_SHARED((N_ACC,), jnp.float32),
    ],
)
def _sc_degree(ei_hbm, out_hbm, col_v, ones_v, zbuf_v, hist_sh):
    cid = lax.axis_index("c")
    sid = lax.axis_index("s")
    wid = sid * NC + cid
    ebase = wid * EW

    def _fill(i, _):
        zbuf_v[pl.ds(i * 16, 16)] = jnp.zeros((16,), jnp.float32)
        return 0

    lax.fori_loop(0, SPAN // 16, _fill, 0)

    def _fill1(i, _):
        ones_v[pl.ds(i * 16, 16)] = jnp.ones((16,), jnp.float32)
        return 0

    lax.fori_loop(0, SLABD // 16, _fill1, 0)

    pltpu.sync_copy(zbuf_v, hist_sh.at[pl.ds(sid * SPAN, SPAN)])
    plsc.subcore_barrier()

    pltpu.sync_copy(ei_hbm.at[1, pl.ds(ebase, JD * SLABD)],
                    col_v.at[pl.ds(0, JD * SLABD)])
    pltpu.sync_copy(ei_hbm.at[1, pl.ds(ebase + JD * SLABD, TW)],
                    col_v.at[pl.ds(JD * SLABD, TW)])

    def _scat(j, _):
        pltpu.sync_copy(ones_v, hist_sh.at[col_v.at[pl.ds(j * SLABD, SLABD)]],
                        add=True)
        return 0

    lax.fori_loop(0, JD, _scat, 0)
    pltpu.sync_copy(ones_v.at[pl.ds(0, TW)],
                    hist_sh.at[col_v.at[pl.ds(JD * SLABD, TW)]], add=True)
    plsc.subcore_barrier()

    pltpu.sync_copy(hist_sh.at[pl.ds(sid * SPAN, SPAN)], zbuf_v)
    pltpu.sync_copy(zbuf_v, out_hbm.at[pl.ds(cid * N_ACC + sid * SPAN, SPAN)])


@functools.partial(
    pl.kernel,
    out_type=jax.ShapeDtypeStruct((NC, N_ACC, HD), jnp.float32),
    mesh=_mesh,
    compiler_params=pltpu.CompilerParams(use_tc_tiling_on_sc=False),
    scratch_types=[
        pltpu.VMEM((ET,), jnp.int32),
        pltpu.VMEM((ET,), jnp.int32),
        [pltpu.VMEM((SLABA, HD), jnp.float32) for _ in range(NBUF)],
        [pltpu.SemaphoreType.DMA for _ in range(NBUF)],
        pltpu.VMEM_SHARED((N_ACC, HD), jnp.float32),
    ],
)
def _sc_aggregate(gl_hbm, gr_hbm, ei_hbm, out_hbm,
                  row_v, col_v, bufs, sems, agg_sh):
    cid = lax.axis_index("c")
    sid = lax.axis_index("s")
    base = sid * SPAN
    ebase = sid * ET

    def _zrow(r, _):
        for i in range(HD // 16):
            bufs[0][r, pl.ds(i * 16, 16)] = jnp.zeros((16,), jnp.float32)
        return 0

    lax.fori_loop(0, SLABA, _zrow, 0)
    for k in range(SPAN // SLABA):
        pltpu.sync_copy(bufs[0], agg_sh.at[pl.ds(base + k * SLABA, SLABA), :])
    rem = SPAN % SLABA
    pltpu.sync_copy(bufs[0].at[pl.ds(0, rem), :],
                    agg_sh.at[pl.ds(base + (SPAN // SLABA) * SLABA, rem), :])
    plsc.subcore_barrier()

    pltpu.sync_copy(ei_hbm.at[0, pl.ds(ebase, JA * SLABA)],
                    row_v.at[pl.ds(0, JA * SLABA)])
    pltpu.sync_copy(ei_hbm.at[0, pl.ds(ebase + JA * SLABA, TA)],
                    row_v.at[pl.ds(JA * SLABA, TA)])
    pltpu.sync_copy(ei_hbm.at[1, pl.ds(ebase, JA * SLABA)],
                    col_v.at[pl.ds(0, JA * SLABA)])
    pltpu.sync_copy(ei_hbm.at[1, pl.ds(ebase + JA * SLABA, TA)],
                    col_v.at[pl.ds(JA * SLABA, TA)])

    def _run(g_hbm):
        for t in range(NBUF):
            pltpu.async_copy(g_hbm.at[row_v.at[pl.ds(t * SLABA, SLABA)]],
                             bufs[t], sems[t])

        def _step(i, _):
            for t in range(NBUF):
                j = i * NBUF + t
                pltpu.make_async_copy(
                    g_hbm.at[row_v.at[pl.ds(0, SLABA)]], bufs[t],
                    sems[t]).wait()
                pltpu.sync_copy(bufs[t],
                                agg_sh.at[col_v.at[pl.ds(j * SLABA, SLABA)]],
                                add=True)

                @pl.when(j + NBUF < JA)
                def _():
                    pltpu.async_copy(
                        g_hbm.at[row_v.at[pl.ds((j + NBUF) * SLABA, SLABA)]],
                        bufs[t], sems[t])
            return 0

        lax.fori_loop(0, JA // NBUF, _step, 0)

        pltpu.sync_copy(g_hbm.at[row_v.at[pl.ds(JA * SLABA, TA)]],
                        bufs[0].at[pl.ds(0, TA), :])
        pltpu.sync_copy(bufs[0].at[pl.ds(0, TA), :],
                        agg_sh.at[col_v.at[pl.ds(JA * SLABA, TA)]], add=True)

    @pl.when(cid == 0)
    def _():
        _run(gl_hbm)

    @pl.when(cid == 1)
    def _():
        _run(gr_hbm)

    plsc.subcore_barrier()
    for k in range(SPAN // SLABA):
        t = k % NBUF
        pltpu.sync_copy(agg_sh.at[pl.ds(base + k * SLABA, SLABA), :], bufs[t])
        pltpu.sync_copy(bufs[t], out_hbm.at[cid, pl.ds(base + k * SLABA,
                                                       SLABA), :])
    pltpu.sync_copy(agg_sh.at[pl.ds(base + (SPAN // SLABA) * SLABA, rem), :],
                    bufs[0].at[pl.ds(0, rem), :])
    pltpu.sync_copy(bufs[0].at[pl.ds(0, rem), :],
                    out_hbm.at[cid, pl.ds(base + (SPAN // SLABA) * SLABA,
                                          rem), :])


def _tc_matmul_body(x_ref, w_ref, h_ref):
    h_ref[...] = jnp.dot(x_ref[...], w_ref[...],
                         preferred_element_type=jnp.float32)


def _tc_transform_body(h_ref, hs_ref, gl_ref, gr_ref):
    dis = lax.rsqrt(hs_ref[...] + 1.0)
    g = h_ref[...] * dis
    gl_ref[...] = g[:, :HD]
    gr_ref[...] = g[:, HD:]


def _tc_bn_body(agg_ref, gl_ref, gr_ref, hs_ref, b_ref, gam_ref,
                bet_ref, out_ref):
    dis = lax.rsqrt(hs_ref[...] + 1.0)
    agg = jnp.concatenate([agg_ref[0, :N, :] + gl_ref[...],
                           agg_ref[1, :N, :] + gr_ref[...]], axis=1)
    pre = dis * agg + b_ref[...]
    mean = jnp.mean(pre, axis=0, keepdims=True)
    d = pre - mean
    var = jnp.mean(d * d, axis=0, keepdims=True)
    out = d * lax.rsqrt(var + 1e-5) * gam_ref[...] + bet_ref[...]
    out_ref[...] = jnp.maximum(out, 0.0)


def kernel(x, edge_index, W, b, gamma, beta):
    hist = _sc_degree(edge_index)
    hs = (hist[:N] + hist[N_ACC:N_ACC + N]).reshape(N, 1)

    h = pl.pallas_call(
        _tc_matmul_body,
        out_shape=jax.ShapeDtypeStruct((N, D), jnp.float32),
    )(x, W)

    gl, gr = pl.pallas_call(
        _tc_transform_body,
        out_shape=(
            jax.ShapeDtypeStruct((N, HD), jnp.float32),
            jax.ShapeDtypeStruct((N, HD), jnp.float32),
        ),
    )(h, hs)

    agg = _sc_aggregate(gl, gr, edge_index)

    out = pl.pallas_call(
        _tc_bn_body,
        out_shape=jax.ShapeDtypeStruct((N, D), jnp.float32),
    )(agg, gl, gr, hs, b.reshape(1, D), gamma.reshape(1, D),
      beta.reshape(1, D))
    return out


# speedup vs baseline: 1.0030x; 1.0030x over previous
import functools

import jax
import jax.numpy as jnp
from jax import lax
from jax.experimental import pallas as pl
from jax.experimental.pallas import tpu as pltpu
from jax.experimental.pallas import tpu_sc as plsc

N = 10000
D = 128
HD = D // 2
E = 320000

NC = 2
NS = 16
NW = NC * NS
CHUNK = 128

EW = E // NW
SLABD = 512
JD = EW // SLABD
TW = EW - JD * SLABD

ET = E // NS
JA = ET // CHUNK
TA = ET - JA * CHUNK
NBUF = 4

SPAN = 640
N_ACC = NS * SPAN

_mesh = plsc.VectorSubcoreMesh(core_axis_name="c", subcore_axis_name="s")


@functools.partial(
    pl.kernel,
    out_type=jax.ShapeDtypeStruct((NC * N_ACC,), jnp.float32),
    mesh=_mesh,
    compiler_params=pltpu.CompilerParams(use_tc_tiling_on_sc=False),
    scratch_types=[
        pltpu.VMEM((EW,), jnp.int32),
        pltpu.VMEM((SLABD,), jnp.float32),
        pltpu.VMEM((SPAN,), jnp.float32),
        pltpu.VMEM_SHARED((N_ACC,), jnp.float32),
    ],
)
def _sc_degree(ei_hbm, out_hbm, col_v, ones_v, zbuf_v, hist_sh):
    cid = lax.axis_index("c")
    sid = lax.axis_index("s")
    wid = sid * NC + cid
    ebase = wid * EW

    def _fill(i, _):
        zbuf_v[pl.ds(i * 16, 16)] = jnp.zeros((16,), jnp.float32)
        return 0

    lax.fori_loop(0, SPAN // 16, _fill, 0)

    def _fill1(i, _):
        ones_v[pl.ds(i * 16, 16)] = jnp.ones((16,), jnp.float32)
        return 0

    lax.fori_loop(0, SLABD // 16, _fill1, 0)

    pltpu.sync_copy(zbuf_v, hist_sh.at[pl.ds(sid * SPAN, SPAN)])
    plsc.subcore_barrier()

    pltpu.sync_copy(ei_hbm.at[1, pl.ds(ebase, JD * SLABD)],
                    col_v.at[pl.ds(0, JD * SLABD)])
    pltpu.sync_copy(ei_hbm.at[1, pl.ds(ebase + JD * SLABD, TW)],
                    col_v.at[pl.ds(JD * SLABD, TW)])

    def _scat(j, _):
        pltpu.sync_copy(ones_v, hist_sh.at[col_v.at[pl.ds(j * SLABD, SLABD)]],
                        add=True)
        return 0

    lax.fori_loop(0, JD, _scat, 0)
    pltpu.sync_copy(ones_v.at[pl.ds(0, TW)],
                    hist_sh.at[col_v.at[pl.ds(JD * SLABD, TW)]], add=True)
    plsc.subcore_barrier()

    pltpu.sync_copy(hist_sh.at[pl.ds(sid * SPAN, SPAN)], zbuf_v)
    pltpu.sync_copy(zbuf_v, out_hbm.at[pl.ds(cid * N_ACC + sid * SPAN, SPAN)])


@functools.partial(
    pl.kernel,
    out_type=jax.ShapeDtypeStruct((NC, N_ACC, HD), jnp.float32),
    mesh=_mesh,
    compiler_params=pltpu.CompilerParams(use_tc_tiling_on_sc=False),
    scratch_types=[
        pltpu.VMEM((ET,), jnp.int32),
        pltpu.VMEM((ET,), jnp.int32),
        [pltpu.VMEM((CHUNK, HD), jnp.float32) for _ in range(NBUF)],
        [pltpu.SemaphoreType.DMA for _ in range(NBUF)],
        pltpu.VMEM_SHARED((N_ACC, HD), jnp.float32),
    ],
)
def _sc_aggregate(gl_hbm, gr_hbm, ei_hbm, out_hbm,
                  row_v, col_v, bufs, sems, agg_sh):
    cid = lax.axis_index("c")
    sid = lax.axis_index("s")
    base = sid * SPAN
    ebase = sid * ET

    def _zrow(r, _):
        for i in range(HD // 16):
            bufs[0][r, pl.ds(i * 16, 16)] = jnp.zeros((16,), jnp.float32)
        return 0

    lax.fori_loop(0, CHUNK, _zrow, 0)
    for k in range(SPAN // CHUNK):
        pltpu.sync_copy(bufs[0], agg_sh.at[pl.ds(base + k * CHUNK, CHUNK), :])
    plsc.subcore_barrier()

    pltpu.sync_copy(ei_hbm.at[0, pl.ds(ebase, JA * CHUNK)],
                    row_v.at[pl.ds(0, JA * CHUNK)])
    pltpu.sync_copy(ei_hbm.at[0, pl.ds(ebase + JA * CHUNK, TA)],
                    row_v.at[pl.ds(JA * CHUNK, TA)])
    pltpu.sync_copy(ei_hbm.at[1, pl.ds(ebase, JA * CHUNK)],
                    col_v.at[pl.ds(0, JA * CHUNK)])
    pltpu.sync_copy(ei_hbm.at[1, pl.ds(ebase + JA * CHUNK, TA)],
                    col_v.at[pl.ds(JA * CHUNK, TA)])

    def _run(g_hbm):
        for t in range(NBUF):
            pltpu.async_copy(g_hbm.at[row_v.at[pl.ds(t * CHUNK, CHUNK)]],
                             bufs[t], sems[t])

        def _step(i, _):
            for t in range(NBUF):
                j = i * NBUF + t
                pltpu.make_async_copy(
                    g_hbm.at[row_v.at[pl.ds(0, CHUNK)]], bufs[t],
                    sems[t]).wait()
                pltpu.sync_copy(bufs[t],
                                agg_sh.at[col_v.at[pl.ds(j * CHUNK, CHUNK)]],
                                add=True)

                @pl.when(j + NBUF < JA)
                def _():
                    pltpu.async_copy(
                        g_hbm.at[row_v.at[pl.ds((j + NBUF) * CHUNK, CHUNK)]],
                        bufs[t], sems[t])
            return 0

        lax.fori_loop(0, JA // NBUF, _step, 0)

        pltpu.sync_copy(g_hbm.at[row_v.at[pl.ds(JA * CHUNK, TA)]],
                        bufs[0].at[pl.ds(0, TA), :])
        pltpu.sync_copy(bufs[0].at[pl.ds(0, TA), :],
                        agg_sh.at[col_v.at[pl.ds(JA * CHUNK, TA)]], add=True)

    @pl.when(cid == 0)
    def _():
        _run(gl_hbm)

    @pl.when(cid == 1)
    def _():
        _run(gr_hbm)

    plsc.subcore_barrier()
    for k in range(SPAN // CHUNK):
        t = k % NBUF
        pltpu.sync_copy(agg_sh.at[pl.ds(base + k * CHUNK, CHUNK), :], bufs[t])
        pltpu.sync_copy(bufs[t], out_hbm.at[cid, pl.ds(base + k * CHUNK,
                                                       CHUNK), :])


def _tc_matmul_body(x_ref, w_ref, h_ref):
    h_ref[...] = jnp.dot(x_ref[...], w_ref[...],
                         preferred_element_type=jnp.float32)


def _tc_transform_body(h_ref, hs_ref, gl_ref, gr_ref):
    dis = lax.rsqrt(hs_ref[...] + 1.0)
    g = h_ref[...] * dis
    gl_ref[...] = g[:, :HD]
    gr_ref[...] = g[:, HD:]


def _tc_bn_body(agg_ref, gl_ref, gr_ref, hs_ref, b_ref, gam_ref,
                bet_ref, out_ref):
    dis = lax.rsqrt(hs_ref[...] + 1.0)
    agg = jnp.concatenate([agg_ref[0, :N, :] + gl_ref[...],
                           agg_ref[1, :N, :] + gr_ref[...]], axis=1)
    pre = dis * agg + b_ref[...]
    mean = jnp.mean(pre, axis=0, keepdims=True)
    d = pre - mean
    var = jnp.mean(d * d, axis=0, keepdims=True)
    out = d * lax.rsqrt(var + 1e-5) * gam_ref[...] + bet_ref[...]
    out_ref[...] = jnp.maximum(out, 0.0)


def kernel(x, edge_index, W, b, gamma, beta):
    hist = _sc_degree(edge_index)
    hs = (hist[:N] + hist[N_ACC:N_ACC + N]).reshape(N, 1)

    h = pl.pallas_call(
        _tc_matmul_body,
        out_shape=jax.ShapeDtypeStruct((N, D), jnp.float32),
    )(x, W)

    gl, gr = pl.pallas_call(
        _tc_transform_body,
        out_shape=(
            jax.ShapeDtypeStruct((N, HD), jnp.float32),
            jax.ShapeDtypeStruct((N, HD), jnp.float32),
        ),
    )(h, hs)

    agg = _sc_aggregate(gl, gr, edge_index)

    out = pl.pallas_call(
        _tc_bn_body,
        out_shape=jax.ShapeDtypeStruct((N, D), jnp.float32),
    )(agg, gl, gr, hs, b.reshape(1, D), gamma.reshape(1, D),
      beta.reshape(1, D))
    return out
